# Initial kernel scaffold; baseline (speedup 1.0000x reference)
import functools

import jax
import jax.numpy as jnp
from jax import lax
from jax.experimental import pallas as pl
from jax.experimental.pallas import tpu as pltpu
from jax.experimental.pallas import tpu_sc as plsc

D = 32
NC = 2
NS = 16
NW = NC * NS
CHUNK = 128
K = 8
GROUP = CHUNK * K


@functools.lru_cache(maxsize=None)
def _make_gather(B: int):
    assert B % (NW * GROUP) == 0
    b_per_w = B // NW
    chunks_per_w = b_per_w // CHUNK
    n_groups = b_per_w // GROUP
    mesh = plsc.VectorSubcoreMesh(core_axis_name="c", subcore_axis_name="s")

    @functools.partial(
        pl.kernel,
        mesh=mesh,
        out_type=jax.ShapeDtypeStruct((B, D), jnp.float32),
        scratch_types=[
            pltpu.VMEM((chunks_per_w, CHUNK), jnp.int32),
            pltpu.VMEM((GROUP, D), jnp.float32),
            pltpu.SemaphoreType.DMA,
        ],
    )
    def gather_kernel(table_hbm, idx_hbm, out_hbm, idx_v, rows_v, sem):
        wid = lax.axis_index("s") * NC + lax.axis_index("c")
        chunk_base = wid * chunks_per_w
        row_base = wid * b_per_w
        pltpu.sync_copy(idx_hbm.at[pl.ds(chunk_base, chunks_per_w)], idx_v)

        @pl.loop(0, n_groups)
        def _(g):
            copies = []
            for j in range(K):
                c = g * K + j
                copies.append(
                    pltpu.async_copy(
                        table_hbm.at[idx_v.at[c]],
                        rows_v.at[pl.ds(j * CHUNK, CHUNK)],
                        sem,
                    )
                )
            for cp in copies:
                cp.wait()
            pltpu.sync_copy(rows_v, out_hbm.at[pl.ds(row_base + g * GROUP, GROUP)])

    return gather_kernel


def kernel(x, table):
    B = x.size
    idx2d = x.reshape(B // CHUNK, CHUNK).astype(jnp.int32)
    out = _make_gather(B)(table, idx2d)
    return out.reshape(x.shape + (D,))


# speedup vs baseline: 1.1027x; 1.1027x over previous
import functools

import jax
import jax.numpy as jnp
from jax import lax
from jax.experimental import pallas as pl
from jax.experimental.pallas import tpu as pltpu
from jax.experimental.pallas import tpu_sc as plsc

D = 32
NC = 2
NS = 16
NW = NC * NS
CHUNK = 128
K = 8
GROUP = CHUNK * K


@functools.lru_cache(maxsize=None)
def _make_gather(B: int):
    assert B % (NW * GROUP) == 0
    b_per_w = B // NW
    chunks_per_w = b_per_w // CHUNK
    n_groups = b_per_w // GROUP
    mesh = plsc.VectorSubcoreMesh(core_axis_name="c", subcore_axis_name="s")

    @functools.partial(
        pl.kernel,
        mesh=mesh,
        out_type=jax.ShapeDtypeStruct((B, D), jnp.float32),
        scratch_types=[
            pltpu.VMEM((chunks_per_w, CHUNK), jnp.int32),
            pltpu.VMEM((GROUP, D), jnp.float32),
            pltpu.SemaphoreType.DMA,
        ],
        compiler_params=pltpu.CompilerParams(use_tc_tiling_on_sc=False),
    )
    def gather_kernel(table_hbm, idx_hbm, out_hbm, idx_v, rows_v, sem):
        wid = lax.axis_index("s") * NC + lax.axis_index("c")
        chunk_base = wid * chunks_per_w
        row_base = wid * b_per_w
        pltpu.sync_copy(idx_hbm.at[pl.ds(chunk_base, chunks_per_w)], idx_v)

        @pl.loop(0, n_groups)
        def _(g):
            copies = []
            for j in range(K):
                c = g * K + j
                copies.append(
                    pltpu.async_copy(
                        table_hbm.at[idx_v.at[c]],
                        rows_v.at[pl.ds(j * CHUNK, CHUNK)],
                        sem,
                    )
                )
            for cp in copies:
                cp.wait()
            pltpu.sync_copy(rows_v, out_hbm.at[pl.ds(row_base + g * GROUP, GROUP)])

    return gather_kernel


def kernel(x, table):
    B = x.size
    idx2d = x.reshape(B // CHUNK, CHUNK).astype(jnp.int32)
    out = _make_gather(B)(table, idx2d)
    return out.reshape(x.shape + (D,))


# speedup vs baseline: 1.1134x; 1.0097x over previous
import functools

import jax
import jax.numpy as jnp
from jax import lax
from jax.experimental import pallas as pl
from jax.experimental.pallas import tpu as pltpu
from jax.experimental.pallas import tpu_sc as plsc

D = 32
NC = 2
NS = 16
NW = NC * NS
CHUNK = 128
K = 8
GROUP = CHUNK * K
NBUF = 2


@functools.lru_cache(maxsize=None)
def _make_gather(B: int):
    assert B % (NW * GROUP) == 0
    b_per_w = B // NW
    chunks_per_w = b_per_w // CHUNK
    n_groups = b_per_w // GROUP
    mesh = plsc.VectorSubcoreMesh(core_axis_name="c", subcore_axis_name="s")

    @functools.partial(
        pl.kernel,
        mesh=mesh,
        out_type=jax.ShapeDtypeStruct((B, D), jnp.float32),
        scratch_types=[
            pltpu.VMEM((chunks_per_w, CHUNK), jnp.int32),
            pltpu.VMEM((NBUF, GROUP, D), jnp.float32),
            pltpu.SemaphoreType.DMA((NBUF,)),
            pltpu.SemaphoreType.DMA((NBUF,)),
        ],
        compiler_params=pltpu.CompilerParams(use_tc_tiling_on_sc=False),
    )
    def gather_kernel(table_hbm, idx_hbm, out_hbm, idx_v, rows_v, gsem, wsem):
        wid = lax.axis_index("s") * NC + lax.axis_index("c")
        chunk_base = wid * chunks_per_w
        row_base = wid * b_per_w
        pltpu.sync_copy(idx_hbm.at[pl.ds(chunk_base, chunks_per_w)], idx_v)

        def fire(g, b):
            for j in range(K):
                pltpu.async_copy(
                    table_hbm.at[idx_v.at[g * K + j]],
                    rows_v.at[b, pl.ds(j * CHUNK, CHUNK)],
                    gsem.at[b],
                )

        def drain_gathers(b):
            for j in range(K):
                pltpu.make_async_copy(
                    table_hbm.at[idx_v.at[j]],
                    rows_v.at[b, pl.ds(j * CHUNK, CHUNK)],
                    gsem.at[b],
                ).wait()

        def wait_write(b):
            pltpu.make_async_copy(
                rows_v.at[b],
                out_hbm.at[pl.ds(row_base, GROUP)],
                wsem.at[b],
            ).wait()

        fire(0, 0)

        @pl.loop(0, n_groups)
        def _(g):
            b = g % NBUF
            nb = (g + 1) % NBUF

            @pl.when(g + 1 < n_groups)
            def _():
                @pl.when(g + 1 >= NBUF)
                def _():
                    wait_write(nb)

                fire(g + 1, nb)

            drain_gathers(b)
            pltpu.async_copy(
                rows_v.at[b],
                out_hbm.at[pl.ds(row_base + g * GROUP, GROUP)],
                wsem.at[b],
            )

        for b in range(NBUF):
            wait_write(b)

    return gather_kernel


def kernel(x, table):
    B = x.size
    idx2d = x.reshape(B // CHUNK, CHUNK).astype(jnp.int32)
    out = _make_gather(B)(table, idx2d)
    return out.reshape(x.shape + (D,))


# speedup vs baseline: 1.6450x; 1.4774x over previous
import functools

import jax
import jax.numpy as jnp
from jax import lax
from jax.experimental import pallas as pl
from jax.experimental.pallas import tpu as pltpu
from jax.experimental.pallas import tpu_sc as plsc

D = 32
NC = 2
NS = 16
NW = NC * NS
CHUNK = 128
NBUF = 2
LANES = 16


@functools.lru_cache(maxsize=None)
def _make_gather(S: int, NB: int):
    n_chunks = S * NB // CHUNK
    assert n_chunks % NW == 0
    chunks_per_w = n_chunks // NW
    cb_per_s = NB // CHUNK
    mesh = plsc.VectorSubcoreMesh(core_axis_name="c", subcore_axis_name="s")

    @functools.partial(
        pl.kernel,
        mesh=mesh,
        out_type=jax.ShapeDtypeStruct((S, D // 8, cb_per_s, 8, CHUNK), jnp.float32),
        scratch_types=[
            pltpu.VMEM((chunks_per_w, CHUNK), jnp.int32),
            pltpu.VMEM((NBUF, CHUNK, D), jnp.float32),
            pltpu.VMEM((NBUF, D // 8, 8, CHUNK), jnp.float32),
            pltpu.SemaphoreType.DMA((NBUF,)),
            pltpu.SemaphoreType.DMA((NBUF,)),
        ],
        compiler_params=pltpu.CompilerParams(
            use_tc_tiling_on_sc=False, needs_layout_passes=False
        ),
    )
    def gather_kernel(table_hbm, idx_hbm, out_hbm, idx_v, rows_v, tile_v, gsem, wsem):
        wid = lax.axis_index("s") * NC + lax.axis_index("c")
        chunk_base = wid * chunks_per_w
        pltpu.sync_copy(idx_hbm.at[pl.ds(chunk_base, chunks_per_w)], idx_v)

        lane_iota = lax.iota(jnp.int32, LANES)

        def fire(g, b):
            pltpu.async_copy(table_hbm.at[idx_v.at[g]], rows_v.at[b], gsem.at[b])

        def drain_gather(b):
            pltpu.make_async_copy(
                table_hbm.at[idx_v.at[0]], rows_v.at[b], gsem.at[b]
            ).wait()

        def wait_writes(b):
            for a in range(D // 8):
                pltpu.make_async_copy(
                    tile_v.at[b, a], out_hbm.at[0, a, 0], wsem.at[b]
                ).wait()

        fire(0, 0)

        @pl.loop(0, chunks_per_w)
        def _(g):
            b = g % NBUF
            nb = (g + 1) % NBUF

            @pl.when(g + 1 < chunks_per_w)
            def _():
                fire(g + 1, nb)

            drain_gather(b)

            @pl.when(g >= NBUF)
            def _():
                wait_writes(b)

            rows_b = rows_v.at[b]
            for d in range(D):
                for b0 in range(0, CHUNK, LANES):
                    vec = plsc.load_gather(
                        rows_b, [lane_iota + b0, jnp.full((LANES,), d, jnp.int32)]
                    )
                    tile_v[b, d // 8, d % 8, pl.ds(b0, LANES)] = vec

            c = chunk_base + g
            s = c // cb_per_s
            cb = c % cb_per_s
            for a in range(D // 8):
                pltpu.async_copy(tile_v.at[b, a], out_hbm.at[s, a, cb], wsem.at[b])

        for b in range(NBUF):
            wait_writes(b)

    return gather_kernel


def kernel(x, table):
    NB, S = x.shape
    idx2d = jnp.transpose(x).reshape(S * NB // CHUNK, CHUNK).astype(jnp.int32)
    out5 = _make_gather(S, NB)(table, idx2d)
    return jnp.transpose(out5, (2, 4, 0, 1, 3)).reshape(NB, S, D)


# speedup vs baseline: 2.4303x; 1.4774x over previous
import functools

import jax
import jax.numpy as jnp
from jax import lax
from jax.experimental import pallas as pl
from jax.experimental.pallas import tpu as pltpu
from jax.experimental.pallas import tpu_sc as plsc

D = 32
NC = 2
NS = 16
NW = NC * NS
CHUNK = 128
NBUF = 2
GDEPTH = 8
LANES = 16


@functools.lru_cache(maxsize=None)
def _make_gather(S: int, NB: int):
    n_chunks = S * NB // CHUNK
    assert n_chunks % NW == 0
    chunks_per_w = n_chunks // NW
    cb_per_s = NB // CHUNK
    mesh = plsc.VectorSubcoreMesh(core_axis_name="c", subcore_axis_name="s")

    @functools.partial(
        pl.kernel,
        mesh=mesh,
        out_type=jax.ShapeDtypeStruct((S, D // 8, cb_per_s, 8, CHUNK), jnp.float32),
        scratch_types=[
            pltpu.VMEM((chunks_per_w, CHUNK), jnp.int32),
            pltpu.VMEM((GDEPTH, CHUNK, D), jnp.float32),
            pltpu.VMEM((NBUF, D, CHUNK + 1), jnp.float32),
            pltpu.SemaphoreType.DMA((GDEPTH,)),
            pltpu.SemaphoreType.DMA((NBUF,)),
        ],
        compiler_params=pltpu.CompilerParams(
            use_tc_tiling_on_sc=False,
            needs_layout_passes=False,
            disable_bounds_checks=True
        ),
    )
    def gather_kernel(table_hbm, idx_hbm, out_hbm, idx_v, rows_v, tile_v, gsem, wsem):
        wid = lax.axis_index("s") * NC + lax.axis_index("c")
        chunk_base = wid * chunks_per_w
        pltpu.sync_copy(idx_hbm.at[pl.ds(chunk_base, chunks_per_w)], idx_v)

        lane_iota = lax.iota(jnp.int32, LANES)

        def fire(g, b):
            pltpu.async_copy(table_hbm.at[idx_v.at[g]], rows_v.at[b], gsem.at[b])

        def drain_gather(b):
            pltpu.make_async_copy(
                table_hbm.at[idx_v.at[0]], rows_v.at[b], gsem.at[b]
            ).wait()

        def wait_writes(b):
            for a in range(D // 8):
                pltpu.make_async_copy(
                    tile_v.at[b, pl.ds(a * 8, 8), pl.ds(0, CHUNK)],
                    out_hbm.at[0, a, 0],
                    wsem.at[b],
                ).wait()

        def transpose_and_write(g, gb, b):
            for r in range(CHUNK):
                colr = jnp.full((LANES,), r, jnp.int32)
                lo = rows_v[gb, r, pl.ds(0, LANES)]
                hi = rows_v[gb, r, pl.ds(LANES, LANES)]
                plsc.store_scatter(tile_v.at[b], [lane_iota, colr], lo)
                plsc.store_scatter(tile_v.at[b], [lane_iota + LANES, colr], hi)
            c = chunk_base + g
            s = c // cb_per_s
            cb = c % cb_per_s
            for a in range(D // 8):
                pltpu.async_copy(
                    tile_v.at[b, pl.ds(a * 8, 8), pl.ds(0, CHUNK)],
                    out_hbm.at[s, a, cb],
                    wsem.at[b],
                )

        for j in range(GDEPTH):
            fire(j, j)

        @pl.loop(0, chunks_per_w, step=GDEPTH)
        def _(g):
            for j in range(GDEPTH):
                drain_gather(j)

                tb = j % NBUF
                @pl.when(g + j >= NBUF)
                def _():
                    wait_writes(tb)

                transpose_and_write(g + j, j, tb)

                nxt = g + j + GDEPTH

                @pl.when(nxt < chunks_per_w)
                def _():
                    fire(nxt, j)

        for b in range(NBUF):
            wait_writes(b)

    return gather_kernel


def kernel(x, table):
    NB, S = x.shape
    idx2d = jnp.transpose(x).reshape(S * NB // CHUNK, CHUNK).astype(jnp.int32)
    out5 = _make_gather(S, NB)(table, idx2d)
    return jnp.transpose(out5, (2, 4, 0, 1, 3)).reshape(NB, S, D)


# speedup vs baseline: 2.4601x; 1.0123x over previous
import functools

import jax
import jax.numpy as jnp
from jax import lax
from jax.experimental import pallas as pl
from jax.experimental.pallas import tpu as pltpu
from jax.experimental.pallas import tpu_sc as plsc

D = 32
NC = 2
NS = 16
NW = NC * NS
CHUNK = 128
NBUF = 2
GDEPTH = 4
LANES = 16


@functools.lru_cache(maxsize=None)
def _make_gather(S: int, NB: int):
    n_chunks = S * NB // CHUNK
    assert n_chunks % NW == 0
    chunks_per_w = n_chunks // NW
    cb_per_s = NB // CHUNK
    mesh = plsc.VectorSubcoreMesh(core_axis_name="c", subcore_axis_name="s")

    @functools.partial(
        pl.kernel,
        mesh=mesh,
        out_type=jax.ShapeDtypeStruct((S, D // 8, cb_per_s, 8, CHUNK), jnp.float32),
        scratch_types=[
            pltpu.VMEM((chunks_per_w, CHUNK), jnp.int32),
            pltpu.VMEM((GDEPTH, CHUNK, 128), jnp.float32),
            pltpu.VMEM((NBUF, D, CHUNK + 1), jnp.float32),
            pltpu.SemaphoreType.DMA((GDEPTH,)),
            pltpu.SemaphoreType.DMA((NBUF,)),
        ],
        compiler_params=pltpu.CompilerParams(
            use_tc_tiling_on_sc=False,
            needs_layout_passes=False,
            disable_bounds_checks=True
        ),
    )
    def gather_kernel(table_hbm, idx_hbm, out_hbm, idx_v, rows_v, tile_v, gsem, wsem):
        wid = lax.axis_index("s") * NC + lax.axis_index("c")
        chunk_base = wid * chunks_per_w
        pltpu.sync_copy(idx_hbm.at[pl.ds(chunk_base, chunks_per_w)], idx_v)

        lane_iota = lax.iota(jnp.int32, LANES)

        def fire(g, b):
            pltpu.async_copy(table_hbm.at[idx_v.at[g]], rows_v.at[b], gsem.at[b])

        def drain_gather(b):
            pltpu.make_async_copy(
                table_hbm.at[idx_v.at[0]], rows_v.at[b], gsem.at[b]
            ).wait()

        def wait_writes(b):
            for a in range(D // 8):
                pltpu.make_async_copy(
                    tile_v.at[b, pl.ds(a * 8, 8), pl.ds(0, CHUNK)],
                    out_hbm.at[0, a, 0],
                    wsem.at[b],
                ).wait()

        def transpose_and_write(g, gb, b):
            for r in range(CHUNK):
                colr = jnp.full((LANES,), r, jnp.int32)
                lo = rows_v[gb, r, pl.ds(0, LANES)]
                hi = rows_v[gb, r, pl.ds(LANES, LANES)]
                plsc.store_scatter(tile_v.at[b], [lane_iota, colr], lo)
                plsc.store_scatter(tile_v.at[b], [lane_iota + LANES, colr], hi)
            c = chunk_base + g
            s = c // cb_per_s
            cb = c % cb_per_s
            for a in range(D // 8):
                pltpu.async_copy(
                    tile_v.at[b, pl.ds(a * 8, 8), pl.ds(0, CHUNK)],
                    out_hbm.at[s, a, cb],
                    wsem.at[b],
                )

        for j in range(GDEPTH):
            fire(j, j)

        @pl.loop(0, chunks_per_w, step=GDEPTH)
        def _(g):
            for j in range(GDEPTH):
                drain_gather(j)

                tb = j % NBUF
                @pl.when(g + j >= NBUF)
                def _():
                    wait_writes(tb)

                transpose_and_write(g + j, j, tb)

                nxt = g + j + GDEPTH

                @pl.when(nxt < chunks_per_w)
                def _():
                    fire(nxt, j)

        for b in range(NBUF):
            wait_writes(b)

    return gather_kernel


def kernel(x, table):
    NB, S = x.shape
    idx2d = jnp.transpose(x).reshape(S * NB // CHUNK, CHUNK).astype(jnp.int32)
    tpad = jnp.pad(table, ((0, 0), (0, 128 - D)))
    out5 = _make_gather(S, NB)(tpad, idx2d)
    return jnp.transpose(out5, (2, 4, 0, 1, 3)).reshape(NB, S, D)
